# baseline (speedup 1.0000x reference)
import jax
import jax.numpy as jnp
from jax import lax
from jax.experimental import pallas as pl
from jax.experimental.pallas import tpu as pltpu
from jax.experimental.pallas import tpu_sc as plsc

N = 10000
D = 128
NPAD = 10112
NC = 2
NS = 16
RPT = NPAD // NS
CHUNK = 128
EPAD = 327680
NCHUNKS = EPAD // CHUNK
BLK = 16
NBLK = NCHUNKS // BLK
CPT = NCHUNKS // (NC * NS)
NB = 2
CPT0 = 144
CPT1 = 16
NBLK0 = CPT0 // BLK
NBLK1 = CPT1 // BLK

_MESH = dict(core_axis_name="c", subcore_axis_name="s",
             num_cores=NC, num_subcores=NS)


def _unpack_chunk(pidx_v, bi, off, sst_v, dst_v, slot):
    for u in range(CHUNK // 16):
        v = pidx_v[bi, off, pl.ds(u * 16, 16)]
        sst_v[slot, pl.ds(u * 16, 16)] = v & 0xFFFF
        dst_v[slot, pl.ds(u * 16, 16)] = lax.shift_right_logical(v, 16)


def _deg_sc(pidx_hbm, z_hbm, ones_hbm, out_hbm, pidx_v, dst_v, ones_v,
            acc_sh):
    c = lax.axis_index("c")
    s = lax.axis_index("s")
    tid = c * NS + s
    pltpu.sync_copy(z_hbm.at[pl.ds(s * RPT, RPT)],
                    acc_sh.at[pl.ds(s * RPT, RPT)])
    pltpu.sync_copy(ones_hbm, ones_v)
    pltpu.sync_copy(pidx_hbm.at[pl.ds(tid * (CPT // BLK), CPT // BLK)],
                    pidx_v)
    plsc.subcore_barrier()

    def body(j, carry):
        bi = lax.div(j, BLK)
        off = lax.rem(j, BLK)
        for u in range(CHUNK // 16):
            v = pidx_v[bi, off, pl.ds(u * 16, 16)]
            dst_v[pl.ds(u * 16, 16)] = lax.shift_right_logical(v, 16)
        pltpu.sync_copy(ones_v, acc_sh.at[dst_v], add=True)
        return carry

    lax.fori_loop(0, CPT, body, 0)
    plsc.subcore_barrier()
    pltpu.sync_copy(acc_sh.at[pl.ds(s * RPT, RPT)],
                    out_hbm.at[c, pl.ds(s * RPT, RPT)])


def _acc_core(nblk, blk0, s, pidx_hbm, h_hbm, acc_sh,
              blk_v, sst_v, dst_v, rows_v, gsems, ssem, bsem):
    def phase(slot, unp):
        pltpu.make_async_copy(h_hbm.at[sst_v.at[slot]], rows_v.at[slot],
                              gsems[slot]).wait()
        pltpu.async_copy(rows_v.at[slot], acc_sh.at[dst_v.at[slot]], ssem,
                         add=True)
        pltpu.make_async_copy(rows_v.at[slot], acc_sh.at[dst_v.at[slot]],
                              ssem).wait()
        if unp is not None:
            buf, off = unp
            _unpack_chunk(blk_v, buf, off, sst_v, dst_v, slot)
            pltpu.async_copy(h_hbm.at[sst_v.at[slot]], rows_v.at[slot],
                             gsems[slot])

    pltpu.make_async_copy(pidx_hbm.at[blk0], blk_v.at[0], bsem).wait()
    if nblk > 1:
        pltpu.async_copy(pidx_hbm.at[blk0 + 1], blk_v.at[1], bsem)
    for i in range(NB):
        _unpack_chunk(blk_v, 0, i, sst_v, dst_v, i)
        pltpu.async_copy(h_hbm.at[sst_v.at[i]], rows_v.at[i], gsems[i])

    for B in range(nblk):
        buf = B % 2
        nbuf = (B + 1) % 2
        last = B == nblk - 1

        def inner(k, carry, buf=buf):
            off = 2 * k
            phase(0, (buf, off + 2))
            phase(1, (buf, off + 3))
            return carry

        lax.fori_loop(0, BLK // 2 - 1, inner, 0)
        if not last:
            pltpu.make_async_copy(pidx_hbm.at[blk0], blk_v.at[nbuf],
                                  bsem).wait()
            if B + 2 < nblk:
                pltpu.async_copy(pidx_hbm.at[blk0 + B + 2], blk_v.at[buf],
                                 bsem)
            phase(0, (nbuf, 0))
            phase(1, (nbuf, 1))
        else:
            phase(0, None)
            phase(1, None)


def _acc_sc(pidx_hbm, h_hbm, z_hbm, out_hbm,
            blk_v, sst_v, dst_v, rows_v, g0, g1, ssem, bsem, acc_sh):
    c = lax.axis_index("c")
    s = lax.axis_index("s")
    gsems = (g0, g1)
    blk0_0 = s * NBLK0
    blk0_1 = NS * NBLK0 + s * NBLK1

    pltpu.sync_copy(z_hbm.at[pl.ds(s * RPT, RPT)],
                    acc_sh.at[pl.ds(s * RPT, RPT)])
    blk0 = jnp.where(c == 0, blk0_0, blk0_1)
    pltpu.async_copy(pidx_hbm.at[blk0], blk_v.at[0], bsem)
    plsc.subcore_barrier()

    @pl.when(c == 0)
    def _():
        _acc_core(NBLK0, blk0_0, s, pidx_hbm, h_hbm, acc_sh,
                  blk_v, sst_v, dst_v, rows_v, gsems, ssem, bsem)

    @pl.when(c == 1)
    def _():
        _acc_core(NBLK1, blk0_1, s, pidx_hbm, h_hbm, acc_sh,
                  blk_v, sst_v, dst_v, rows_v, gsems, ssem, bsem)

    plsc.subcore_barrier()
    pltpu.sync_copy(acc_sh.at[pl.ds(s * RPT, RPT)],
                    out_hbm.at[c, pl.ds(s * RPT, RPT)])


def _run_deg(pidx_p, z128, ones128):
    return pl.kernel(
        _deg_sc,
        out_type=jax.ShapeDtypeStruct((NC, NPAD, D), jnp.float32),
        mesh=plsc.VectorSubcoreMesh(**_MESH),
        scratch_types=[
            pltpu.VMEM((CPT // BLK, BLK, CHUNK), jnp.int32),
            pltpu.VMEM((CHUNK,), jnp.int32),
            pltpu.VMEM((CHUNK, D), jnp.float32),
            pltpu.VMEM_SHARED((NPAD, D), jnp.float32),
        ],
    )(pidx_p, z128, ones128)


def _run_acc(pidx_p, h, z128):
    return pl.kernel(
        _acc_sc,
        out_type=jax.ShapeDtypeStruct((NC, NPAD, D), jnp.float32),
        mesh=plsc.VectorSubcoreMesh(**_MESH),
        scratch_types=[
            pltpu.VMEM((2, BLK, CHUNK), jnp.int32),
            pltpu.VMEM((NB, CHUNK), jnp.int32),
            pltpu.VMEM((NB, CHUNK), jnp.int32),
            pltpu.VMEM((NB, CHUNK, D), jnp.float32),
            pltpu.SemaphoreType.DMA,
            pltpu.SemaphoreType.DMA,
            pltpu.SemaphoreType.DMA,
            pltpu.SemaphoreType.DMA,
            pltpu.VMEM_SHARED((NPAD, D), jnp.float32),
        ],
    )(pidx_p, h, z128)


BR = 400
GRID = N // BR


def _dinv(d0_ref, d1_ref):
    deg = d0_ref[0, :, 0:1] + d1_ref[0, :, 0:1] + 1.0
    return lax.rsqrt(deg)


def _mm(a, w):
    return lax.dot_general(a, w, (((1,), (0,)), ((), ())),
                           precision=lax.Precision.HIGHEST,
                           preferred_element_type=jnp.float32)


def _h1p_body(x_ref, w_ref, d0_ref, d1_ref, o_ref):
    o_ref[...] = _mm(x_ref[...], w_ref[...]) * _dinv(d0_ref, d1_ref)


def _h2p_body(p0_ref, p1_ref, h_ref, d0_ref, d1_ref, b_ref, w_ref, o_ref):
    dinv = _dinv(d0_ref, d1_ref)
    acc = p0_ref[0] + p1_ref[0] + h_ref[...]
    z = jnp.maximum(acc * dinv + b_ref[...], 0.0)
    o_ref[...] = _mm(z, w_ref[...]) * dinv


def _out_body(q0_ref, q1_ref, h_ref, d0_ref, d1_ref, b_ref, o_ref):
    dinv = _dinv(d0_ref, d1_ref)
    o_ref[...] = (q0_ref[0] + q1_ref[0] + h_ref[...]) * dinv + b_ref[...]


_row_spec = pl.BlockSpec((BR, D), lambda i: (i, 0))
_w_spec = pl.BlockSpec((D, D), lambda i: (0, 0))
_b_spec = pl.BlockSpec((1, D), lambda i: (0, 0))
_deg0_spec = pl.BlockSpec((1, BR, D), lambda i: (0, i, 0))
_deg1_spec = pl.BlockSpec((1, BR, D), lambda i: (1, i, 0))
_p0_spec = pl.BlockSpec((1, BR, D), lambda i: (0, i, 0))
_p1_spec = pl.BlockSpec((1, BR, D), lambda i: (1, i, 0))
_out_sds = jax.ShapeDtypeStruct((N, D), jnp.float32)


def kernel(x, edge_index, W1, b1, W2, b2):
    e = edge_index.shape[1]
    pad = EPAD - e
    packed = edge_index[0] | (edge_index[1] << 16)
    packed = jnp.concatenate(
        [packed, jnp.full((pad,), N << 16, jnp.int32)]
    ).reshape(NBLK, BLK, CHUNK)
    z128 = jnp.zeros((NPAD, D), jnp.float32)
    ones128 = jnp.ones((CHUNK, D), jnp.float32)
    b1r = b1.reshape(1, D)
    b2r = b2.reshape(1, D)

    degp = _run_deg(packed, z128, ones128)

    h1p = pl.pallas_call(
        _h1p_body,
        grid=(GRID,),
        in_specs=[_row_spec, _w_spec, _deg0_spec, _deg1_spec],
        out_specs=_row_spec,
        out_shape=_out_sds,
    )(x, W1, degp, degp)

    p = _run_acc(packed, h1p, z128)

    h2p = pl.pallas_call(
        _h2p_body,
        grid=(GRID,),
        in_specs=[_p0_spec, _p1_spec, _row_spec, _deg0_spec, _deg1_spec,
                  _b_spec, _w_spec],
        out_specs=_row_spec,
        out_shape=_out_sds,
    )(p, p, h1p, degp, degp, b1r, W2)

    q = _run_acc(packed, h2p, z128)

    out = pl.pallas_call(
        _out_body,
        grid=(GRID,),
        in_specs=[_p0_spec, _p1_spec, _row_spec, _deg0_spec, _deg1_spec,
                  _b_spec],
        out_specs=_row_spec,
        out_shape=_out_sds,
    )(q, q, h2p, degp, degp, b2r)

    return out


# speedup vs baseline: 1.1332x; 1.1332x over previous
import jax
import jax.numpy as jnp
from jax import lax
from jax.experimental import pallas as pl
from jax.experimental.pallas import tpu as pltpu
from jax.experimental.pallas import tpu_sc as plsc

N = 10000
D = 128
NPAD = 10112
NC = 2
NS = 16
RPT = NPAD // NS
CHUNK = 128
EPAD = 323584
NCHUNKS = EPAD // CHUNK
CPT = NCHUNKS // (NC * NS)

_MESH = dict(core_axis_name="c", subcore_axis_name="s",
             num_cores=NC, num_subcores=NS)


def _deg_sc(dst_hbm, z_hbm, ones_hbm, out_hbm, idx_v, ones_v, acc_sh):
    c = lax.axis_index("c")
    s = lax.axis_index("s")
    tid = c * NS + s
    pltpu.sync_copy(z_hbm.at[pl.ds(s * RPT, RPT)],
                    acc_sh.at[pl.ds(s * RPT, RPT)])
    pltpu.sync_copy(ones_hbm, ones_v)
    pltpu.sync_copy(dst_hbm.at[tid], idx_v)
    plsc.subcore_barrier()

    def body(j, carry):
        pltpu.sync_copy(ones_v, acc_sh.at[idx_v.at[j]], add=True)
        return carry

    lax.fori_loop(0, CPT, body, 0)
    plsc.subcore_barrier()
    pltpu.sync_copy(acc_sh.at[pl.ds(s * RPT, RPT)],
                    out_hbm.at[c, pl.ds(s * RPT, RPT)])


def _acc_sc(src_hbm, dst_hbm, h_hbm, z_hbm, out_hbm,
            sidx_v, didx_v, rows_v, gsem, acc_sh):
    c = lax.axis_index("c")
    s = lax.axis_index("s")
    tid = c * NS + s
    pltpu.sync_copy(z_hbm.at[pl.ds(s * RPT, RPT)],
                    acc_sh.at[pl.ds(s * RPT, RPT)])
    pltpu.sync_copy(src_hbm.at[tid], sidx_v)
    pltpu.sync_copy(dst_hbm.at[tid], didx_v)
    plsc.subcore_barrier()

    def body(j, carry):
        pltpu.async_copy(h_hbm.at[sidx_v.at[j]], rows_v, gsem).wait()
        pltpu.sync_copy(rows_v, acc_sh.at[didx_v.at[j]], add=True)
        return carry

    lax.fori_loop(0, CPT, body, 0)
    plsc.subcore_barrier()
    pltpu.sync_copy(acc_sh.at[pl.ds(s * RPT, RPT)],
                    out_hbm.at[c, pl.ds(s * RPT, RPT)])


def _run_deg(dst_p, z128, ones128):
    return pl.kernel(
        _deg_sc,
        out_type=jax.ShapeDtypeStruct((NC, NPAD, D), jnp.float32),
        mesh=plsc.VectorSubcoreMesh(**_MESH),
        scratch_types=[
            pltpu.VMEM((CPT, CHUNK), jnp.int32),
            pltpu.VMEM((CHUNK, D), jnp.float32),
            pltpu.VMEM_SHARED((NPAD, D), jnp.float32),
        ],
    )(dst_p, z128, ones128)


def _run_acc(src_p, dst_p, h, z128):
    return pl.kernel(
        _acc_sc,
        out_type=jax.ShapeDtypeStruct((NC, NPAD, D), jnp.float32),
        mesh=plsc.VectorSubcoreMesh(**_MESH),
        scratch_types=[
            pltpu.VMEM((CPT, CHUNK), jnp.int32),
            pltpu.VMEM((CPT, CHUNK), jnp.int32),
            pltpu.VMEM((CHUNK, D), jnp.float32),
            pltpu.SemaphoreType.DMA,
            pltpu.VMEM_SHARED((NPAD, D), jnp.float32),
        ],
    )(src_p, dst_p, h, z128)


BR = 400
GRID = N // BR


def _dinv(d0_ref, d1_ref):
    deg = d0_ref[0, :, 0:1] + d1_ref[0, :, 0:1] + 1.0
    return lax.rsqrt(deg)


def _mm(a, w):
    return lax.dot_general(a, w, (((1,), (0,)), ((), ())),
                           precision=lax.Precision.HIGHEST,
                           preferred_element_type=jnp.float32)


def _h1p_body(x_ref, w_ref, d0_ref, d1_ref, o_ref):
    o_ref[...] = _mm(x_ref[...], w_ref[...]) * _dinv(d0_ref, d1_ref)


def _h2p_body(p0_ref, p1_ref, h_ref, d0_ref, d1_ref, b_ref, w_ref, o_ref):
    dinv = _dinv(d0_ref, d1_ref)
    acc = p0_ref[0] + p1_ref[0] + h_ref[...]
    z = jnp.maximum(acc * dinv + b_ref[...], 0.0)
    o_ref[...] = _mm(z, w_ref[...]) * dinv


def _out_body(q0_ref, q1_ref, h_ref, d0_ref, d1_ref, b_ref, o_ref):
    dinv = _dinv(d0_ref, d1_ref)
    o_ref[...] = (q0_ref[0] + q1_ref[0] + h_ref[...]) * dinv + b_ref[...]


_row_spec = pl.BlockSpec((BR, D), lambda i: (i, 0))
_w_spec = pl.BlockSpec((D, D), lambda i: (0, 0))
_b_spec = pl.BlockSpec((1, D), lambda i: (0, 0))
_deg0_spec = pl.BlockSpec((1, BR, D), lambda i: (0, i, 0))
_deg1_spec = pl.BlockSpec((1, BR, D), lambda i: (1, i, 0))
_p0_spec = pl.BlockSpec((1, BR, D), lambda i: (0, i, 0))
_p1_spec = pl.BlockSpec((1, BR, D), lambda i: (1, i, 0))
_out_sds = jax.ShapeDtypeStruct((N, D), jnp.float32)


def kernel(x, edge_index, W1, b1, W2, b2):
    e = edge_index.shape[1]
    pad = EPAD - e
    src_p = jnp.concatenate(
        [edge_index[0], jnp.zeros((pad,), jnp.int32)]
    ).reshape(NC * NS, CPT, CHUNK)
    dst_p = jnp.concatenate(
        [edge_index[1], jnp.full((pad,), N, jnp.int32)]
    ).reshape(NC * NS, CPT, CHUNK)
    z128 = jnp.zeros((NPAD, D), jnp.float32)
    ones128 = jnp.ones((CHUNK, D), jnp.float32)
    b1r = b1.reshape(1, D)
    b2r = b2.reshape(1, D)

    degp = _run_deg(dst_p, z128, ones128)

    h1p = pl.pallas_call(
        _h1p_body,
        grid=(GRID,),
        in_specs=[_row_spec, _w_spec, _deg0_spec, _deg1_spec],
        out_specs=_row_spec,
        out_shape=_out_sds,
    )(x, W1, degp, degp)

    p = _run_acc(src_p, dst_p, h1p, z128)

    h2p = pl.pallas_call(
        _h2p_body,
        grid=(GRID,),
        in_specs=[_p0_spec, _p1_spec, _row_spec, _deg0_spec, _deg1_spec,
                  _b_spec, _w_spec],
        out_specs=_row_spec,
        out_shape=_out_sds,
    )(p, p, h1p, degp, degp, b1r, W2)

    q = _run_acc(src_p, dst_p, h2p, z128)

    out = pl.pallas_call(
        _out_body,
        grid=(GRID,),
        in_specs=[_p0_spec, _p1_spec, _row_spec, _deg0_spec, _deg1_spec,
                  _b_spec],
        out_specs=_row_spec,
        out_shape=_out_sds,
    )(q, q, h2p, degp, degp, b2r)

    return out
